# Initial kernel scaffold; baseline (speedup 1.0000x reference)
import functools

import jax
import jax.numpy as jnp
from jax import lax
from jax.experimental import pallas as pl
from jax.experimental.pallas import tpu as pltpu
from jax.experimental.pallas import tpu_sc as plsc

N = 10000
E = 160000
D = 256
OUT = 256
H = 128
NC = 2
NS = 16
EPT = E // NS
C = 80
KT = EPT // C
RT = N // NS


def _sc_agg_body(feat2, gidx, sidx, zrow, zdeg, orow,
                 sums, deg, acc, dacc, gv, sv, rows, ones_v, sem):
    c = lax.axis_index("c")
    s = lax.axis_index("s")
    pltpu.sync_copy(orow, ones_v)
    for d in range(2):
        pltpu.sync_copy(zrow, acc.at[pl.ds(s * RT, RT)])

        @pl.when(c == d)
        def _():
            pltpu.sync_copy(zdeg, dacc.at[pl.ds(s * RT, RT)])

        plsc.subcore_barrier()

        pltpu.sync_copy(gidx.at[d, c, pl.ds(s * KT, KT)], gv)
        pltpu.sync_copy(sidx.at[d, pl.ds(s * KT, KT)], sv)

        def chunk(k, carry):
            pltpu.async_copy(feat2.at[gv.at[k]], rows, sem).wait()
            pltpu.sync_copy(rows, acc.at[sv.at[k]], add=True)

            @pl.when(c == d)
            def _():
                pltpu.sync_copy(ones_v, dacc.at[sv.at[k]], add=True)

            return carry

        lax.fori_loop(0, KT, chunk, 0)
        plsc.subcore_barrier()

        pltpu.sync_copy(acc.at[pl.ds(s * RT, RT)],
                        sums.at[d, c, pl.ds(s * RT, RT)])

        @pl.when(c == d)
        def _():
            pltpu.sync_copy(dacc.at[pl.ds(s * RT, RT)],
                            deg.at[d, pl.ds(s * RT, RT)])


_sc_agg = functools.partial(
    pl.kernel,
    out_type=[
        jax.ShapeDtypeStruct((2, NC, N, H), jnp.float32),
        jax.ShapeDtypeStruct((2, N, 16), jnp.float32),
    ],
    mesh=plsc.VectorSubcoreMesh(core_axis_name="c", subcore_axis_name="s"),
    scratch_types=[
        pltpu.VMEM_SHARED((N, H), jnp.float32),
        pltpu.VMEM_SHARED((N, 16), jnp.float32),
        pltpu.VMEM((KT, C), jnp.int32),
        pltpu.VMEM((KT, C), jnp.int32),
        pltpu.VMEM((C, H), jnp.float32),
        pltpu.VMEM((C, 16), jnp.float32),
        pltpu.SemaphoreType.DMA,
    ],
)(_sc_agg_body)


R = 2000


def _tc_body(fl, fr, bl, br, di, do_, wa, wb, wc, wd, out):
    sin = 1.0 / jnp.maximum(di[:, 0:1], 1.0)
    so = 1.0 / jnp.maximum(do_[:, 0:1], 1.0)
    acc = jnp.dot(fl[:] * sin, wa[:], preferred_element_type=jnp.float32)
    acc += jnp.dot(fr[:] * sin, wb[:], preferred_element_type=jnp.float32)
    acc += jnp.dot(bl[:] * so, wc[:], preferred_element_type=jnp.float32)
    acc += jnp.dot(br[:] * so, wd[:], preferred_element_type=jnp.float32)
    out[:] = acc


def _tc_mm(fl, fr, bl, br, di, do_, wa, wb, wc, wd):
    sblock = pl.BlockSpec((R, H), lambda i: (i, 0))
    dblock = pl.BlockSpec((R, 16), lambda i: (i, 0))
    wblock = pl.BlockSpec((H, OUT), lambda i: (0, 0))
    return pl.pallas_call(
        _tc_body,
        grid=(N // R,),
        in_specs=[sblock, sblock, sblock, sblock, dblock, dblock,
                  wblock, wblock, wblock, wblock],
        out_specs=pl.BlockSpec((R, OUT), lambda i: (i, 0)),
        out_shape=jax.ShapeDtypeStruct((N, OUT), jnp.float32),
    )(fl, fr, bl, br, di, do_, wa, wb, wc, wd)


def kernel(feat, edge_index, W):
    feat = feat.astype(jnp.float32)
    ei = edge_index.astype(jnp.int32)
    feat2 = jnp.concatenate([feat[:, :H], feat[:, H:]], axis=0)
    gidx = (ei[:, None, :] + (jnp.arange(NC, dtype=jnp.int32) * N)[None, :, None]
            ).reshape(2, NC, E // C, C)
    sidx = jnp.flip(ei, 0).reshape(2, E // C, C)
    zrow = jnp.zeros((RT, H), jnp.float32)
    zdeg = jnp.zeros((RT, 16), jnp.float32)
    orow = jnp.ones((C, 16), jnp.float32)

    sums, deg = _sc_agg(feat2, gidx, sidx, zrow, zdeg, orow)

    Wt = W.astype(jnp.float32).T
    return _tc_mm(sums[0, 0], sums[0, 1], sums[1, 0], sums[1, 1],
                  deg[0], deg[1],
                  Wt[0:H], Wt[H:D], Wt[D:D + H], Wt[D + H:2 * D])


# speedup vs baseline: 2.3276x; 2.3276x over previous
import functools

import jax
import jax.numpy as jnp
from jax import lax
from jax.experimental import pallas as pl
from jax.experimental.pallas import tpu as pltpu
from jax.experimental.pallas import tpu_sc as plsc

N = 10000
E = 160000
D = 256
OUT = 256
H = 128
NC = 2
NS = 16
EPT = E // NS
C = 80
KT = EPT // C
NP = 10240
RT = NP // NS
L = 16


def _sc_dir_body(feat2, gidx, sidx, zrow, orow, sums, deg,
                 acc, gv, sv, rows, ones_v, sem):
    c = lax.axis_index("c")
    s = lax.axis_index("s")
    pltpu.sync_copy(orow, ones_v)
    pltpu.sync_copy(zrow, acc.at[pl.ds(s * RT, RT)])
    plsc.subcore_barrier()

    def chunk(k, carry):
        pltpu.sync_copy(gidx.at[c, s, k, 0], gv)
        pltpu.sync_copy(sidx.at[s, k, 0], sv)
        pltpu.async_copy(feat2.at[gv], rows, sem).wait()
        pltpu.sync_copy(rows, acc.at[sv], add=True)
        return carry

    lax.fori_loop(0, KT, chunk, 0)
    plsc.subcore_barrier()
    pltpu.sync_copy(acc.at[pl.ds(s * RT, RT)],
                    sums.at[c, pl.ds(s * RT, RT)])
    plsc.subcore_barrier()

    pltpu.sync_copy(zrow, acc.at[pl.ds(s * RT, RT)])
    plsc.subcore_barrier()

    def dchunk(k, carry):
        pltpu.sync_copy(sidx.at[s, k, 0], sv)
        pltpu.sync_copy(ones_v, acc.at[sv], add=True)
        return carry

    lax.fori_loop(0, KT, dchunk, 0)
    plsc.subcore_barrier()
    pltpu.sync_copy(acc.at[pl.ds(s * RT, RT)],
                    deg.at[c, pl.ds(s * RT, RT)])


_sc_dir = functools.partial(
    pl.kernel,
    out_type=[
        jax.ShapeDtypeStruct((NC, NP, H), jnp.float32),
        jax.ShapeDtypeStruct((NC, NP, H), jnp.float32),
    ],
    mesh=plsc.VectorSubcoreMesh(core_axis_name="c", subcore_axis_name="s"),
    scratch_types=[
        pltpu.VMEM_SHARED((NP, H), jnp.float32),
        pltpu.VMEM((C,), jnp.int32),
        pltpu.VMEM((C,), jnp.int32),
        pltpu.VMEM((C, H), jnp.float32),
        pltpu.VMEM((C, H), jnp.float32),
        pltpu.SemaphoreType.DMA,
    ],
)(_sc_dir_body)


R = 2000


def _tc_body(fl, fr, bl, br, di, do_, wa, wb, wc, wd, out):
    sin = 1.0 / jnp.maximum(di[:, 0:1], 1.0)
    so = 1.0 / jnp.maximum(do_[:, 0:1], 1.0)
    acc = jnp.dot(fl[:] * sin, wa[:], preferred_element_type=jnp.float32)
    acc += jnp.dot(fr[:] * sin, wb[:], preferred_element_type=jnp.float32)
    acc += jnp.dot(bl[:] * so, wc[:], preferred_element_type=jnp.float32)
    acc += jnp.dot(br[:] * so, wd[:], preferred_element_type=jnp.float32)
    out[:] = acc


def _tc_mm(fl, fr, bl, br, di, do_, wa, wb, wc, wd):
    sblock = pl.BlockSpec((R, H), lambda i: (i, 0))
    dblock = pl.BlockSpec((R, 16), lambda i: (i, 0))
    wblock = pl.BlockSpec((H, OUT), lambda i: (0, 0))
    return pl.pallas_call(
        _tc_body,
        grid=(N // R,),
        in_specs=[sblock, sblock, sblock, sblock, dblock, dblock,
                  wblock, wblock, wblock, wblock],
        out_specs=pl.BlockSpec((R, OUT), lambda i: (i, 0)),
        out_shape=jax.ShapeDtypeStruct((N, OUT), jnp.float32),
    )(fl, fr, bl, br, di, do_, wa, wb, wc, wd)


def kernel(feat, edge_index, W):
    feat = feat.astype(jnp.float32)
    ei = edge_index.astype(jnp.int32)
    feat2 = jnp.concatenate([feat[:, :H], feat[:, H:]], axis=0)
    offs = (jnp.arange(NC, dtype=jnp.int32) * N)[:, None]
    zrow = jnp.zeros((RT, H), jnp.float32)
    orow = jnp.ones((C, H), jnp.float32)

    outs = []
    for d in range(2):
        gidx = (ei[d][None, :] + offs).reshape(NC, NS, KT, 1, C)
        sidx = ei[1 - d].reshape(NS, KT, 1, C)
        outs.append(_sc_dir(feat2, gidx, sidx, zrow, orow))
    (fsum, fdeg), (bsum, bdeg) = outs

    di = fdeg[0, :N, :16]
    do_ = bdeg[0, :N, :16]
    Wt = W.astype(jnp.float32).T
    return _tc_mm(fsum[0, :N], fsum[1, :N], bsum[0, :N], bsum[1, :N],
                  di, do_,
                  Wt[0:H], Wt[H:D], Wt[D:D + H], Wt[D + H:2 * D])


# speedup vs baseline: 3.1322x; 1.3457x over previous
import functools

import jax
import jax.numpy as jnp
from jax import lax
from jax.experimental import pallas as pl
from jax.experimental.pallas import tpu as pltpu
from jax.experimental.pallas import tpu_sc as plsc

N = 10000
E = 160000
D = 256
OUT = 256
H = 128
NC = 2
NS = 16
EPT = E // NS
C = 80
KT = EPT // C
NP = 10240
RT = NP // NS
L = 16


def _sc_dir_body(feat2, gidx, sidx, zrow, orow, sums, deg,
                 acc, gv0, sv0, gv1, sv1, rows0, rows1, ones_v, g0, g1):
    c = lax.axis_index("c")
    s = lax.axis_index("s")
    pltpu.sync_copy(orow, ones_v)
    pltpu.sync_copy(zrow, acc.at[pl.ds(s * RT, RT)])
    plsc.subcore_barrier()

    pltpu.sync_copy(gidx.at[c, s, 0, 0], gv0)
    pltpu.sync_copy(sidx.at[s, 0, 0], sv0)
    pltpu.async_copy(feat2.at[gv0], rows0, g0)
    pltpu.sync_copy(gidx.at[c, s, 1, 0], gv1)
    pltpu.sync_copy(sidx.at[s, 1, 0], sv1)
    pltpu.async_copy(feat2.at[gv1], rows1, g1)

    def pair(j, carry):
        k0 = 2 * j
        pltpu.make_async_copy(feat2.at[gv0], rows0, g0).wait()
        pltpu.sync_copy(rows0, acc.at[sv0], add=True)
        kk0 = jnp.minimum(k0 + 2, KT - 1)
        pltpu.sync_copy(gidx.at[c, s, kk0, 0], gv0)
        pltpu.sync_copy(sidx.at[s, kk0, 0], sv0)
        pltpu.async_copy(feat2.at[gv0], rows0, g0)
        pltpu.make_async_copy(feat2.at[gv1], rows1, g1).wait()
        pltpu.sync_copy(rows1, acc.at[sv1], add=True)
        kk1 = jnp.minimum(k0 + 3, KT - 1)
        pltpu.sync_copy(gidx.at[c, s, kk1, 0], gv1)
        pltpu.sync_copy(sidx.at[s, kk1, 0], sv1)
        pltpu.async_copy(feat2.at[gv1], rows1, g1)
        return carry

    lax.fori_loop(0, (KT - 1) // 2, pair, 0)
    pltpu.make_async_copy(feat2.at[gv0], rows0, g0).wait()
    pltpu.sync_copy(rows0, acc.at[sv0], add=True)
    pltpu.make_async_copy(feat2.at[gv1], rows1, g1).wait()
    plsc.subcore_barrier()
    pltpu.sync_copy(acc.at[pl.ds(s * RT, RT)],
                    sums.at[c, pl.ds(s * RT, RT)])
    plsc.subcore_barrier()

    pltpu.sync_copy(zrow, acc.at[pl.ds(s * RT, RT)])
    plsc.subcore_barrier()

    def dchunk(k, carry):
        pltpu.sync_copy(sidx.at[s, k, 0], sv0)
        pltpu.sync_copy(ones_v, acc.at[sv0], add=True)
        return carry

    lax.fori_loop(0, KT, dchunk, 0)
    plsc.subcore_barrier()
    pltpu.sync_copy(acc.at[pl.ds(s * RT, RT)],
                    deg.at[c, pl.ds(s * RT, RT)])


_sc_dir = functools.partial(
    pl.kernel,
    out_type=[
        jax.ShapeDtypeStruct((NC, NP, H), jnp.float32),
        jax.ShapeDtypeStruct((NC, NP, H), jnp.float32),
    ],
    mesh=plsc.VectorSubcoreMesh(core_axis_name="c", subcore_axis_name="s"),
    scratch_types=[
        pltpu.VMEM_SHARED((NP, H), jnp.float32),
        pltpu.VMEM((C,), jnp.int32),
        pltpu.VMEM((C,), jnp.int32),
        pltpu.VMEM((C,), jnp.int32),
        pltpu.VMEM((C,), jnp.int32),
        pltpu.VMEM((C, H), jnp.float32),
        pltpu.VMEM((C, H), jnp.float32),
        pltpu.VMEM((C, H), jnp.float32),
        pltpu.SemaphoreType.DMA,
        pltpu.SemaphoreType.DMA,
    ],
)(_sc_dir_body)


R = 2000


def _tc_body(fl, fr, bl, br, di, do_, wa, wb, wc, wd, out):
    sin = 1.0 / jnp.maximum(di[:, 0:1], 1.0)
    so = 1.0 / jnp.maximum(do_[:, 0:1], 1.0)
    acc = jnp.dot(fl[:] * sin, wa[:], preferred_element_type=jnp.float32)
    acc += jnp.dot(fr[:] * sin, wb[:], preferred_element_type=jnp.float32)
    acc += jnp.dot(bl[:] * so, wc[:], preferred_element_type=jnp.float32)
    acc += jnp.dot(br[:] * so, wd[:], preferred_element_type=jnp.float32)
    out[:] = acc


def _tc_mm(fl, fr, bl, br, di, do_, wa, wb, wc, wd):
    sblock = pl.BlockSpec((R, H), lambda i: (i, 0))
    dblock = pl.BlockSpec((R, 16), lambda i: (i, 0))
    wblock = pl.BlockSpec((H, OUT), lambda i: (0, 0))
    return pl.pallas_call(
        _tc_body,
        grid=(N // R,),
        in_specs=[sblock, sblock, sblock, sblock, dblock, dblock,
                  wblock, wblock, wblock, wblock],
        out_specs=pl.BlockSpec((R, OUT), lambda i: (i, 0)),
        out_shape=jax.ShapeDtypeStruct((N, OUT), jnp.float32),
    )(fl, fr, bl, br, di, do_, wa, wb, wc, wd)


def kernel(feat, edge_index, W):
    feat = feat.astype(jnp.float32)
    ei = edge_index.astype(jnp.int32)
    feat2 = jnp.concatenate([feat[:, :H], feat[:, H:]], axis=0)
    offs = (jnp.arange(NC, dtype=jnp.int32) * N)[:, None]
    zrow = jnp.zeros((RT, H), jnp.float32)
    orow = jnp.ones((C, H), jnp.float32)

    outs = []
    for d in range(2):
        gidx = (ei[d][None, :] + offs).reshape(NC, NS, KT, 1, C)
        sidx = ei[1 - d].reshape(NS, KT, 1, C)
        outs.append(_sc_dir(feat2, gidx, sidx, zrow, orow))
    (fsum, fdeg), (bsum, bdeg) = outs

    di = fdeg[0, :N, :16]
    do_ = bdeg[0, :N, :16]
    Wt = W.astype(jnp.float32).T
    return _tc_mm(fsum[0, :N], fsum[1, :N], bsum[0, :N], bsum[1, :N],
                  di, do_,
                  Wt[0:H], Wt[H:D], Wt[D:D + H], Wt[D + H:2 * D])


# speedup vs baseline: 3.6685x; 1.1712x over previous
import functools

import jax
import jax.numpy as jnp
from jax import lax
from jax.experimental import pallas as pl
from jax.experimental.pallas import tpu as pltpu
from jax.experimental.pallas import tpu_sc as plsc

N = 10000
E = 160000
D = 256
OUT = 256
H = 128
NC = 2
NS = 16
EPT = E // NS
C = 80
KT = EPT // C
NP = 10240
RT = NP // NS
L = 16


def _sc_dir_body(feat2, gidx, sidx, zrow, orow, sums, deg,
                 acc, gv0, sv0, gv1, sv1, rows0, rows1, ones_v, g0, g1):
    c = lax.axis_index("c")
    s = lax.axis_index("s")
    pltpu.sync_copy(orow, ones_v)
    pltpu.sync_copy(zrow, acc.at[pl.ds(s * RT, RT)])
    plsc.subcore_barrier()

    pltpu.sync_copy(gidx.at[c, s, 0, 0], gv0)
    pltpu.sync_copy(sidx.at[s, 0, 0], sv0)
    pltpu.async_copy(feat2.at[gv0], rows0, g0)
    pltpu.sync_copy(gidx.at[c, s, 1, 0], gv1)
    pltpu.sync_copy(sidx.at[s, 1, 0], sv1)
    pltpu.async_copy(feat2.at[gv1], rows1, g1)

    def pair(j, carry):
        k0 = 2 * j
        pltpu.make_async_copy(feat2.at[gv0], rows0, g0).wait()
        pltpu.sync_copy(rows0, acc.at[sv0], add=True)
        kk0 = jnp.minimum(k0 + 2, KT - 1)
        pltpu.sync_copy(gidx.at[c, s, kk0, 0], gv0)
        pltpu.sync_copy(sidx.at[s, kk0, 0], sv0)
        pltpu.async_copy(feat2.at[gv0], rows0, g0)
        pltpu.make_async_copy(feat2.at[gv1], rows1, g1).wait()
        pltpu.sync_copy(rows1, acc.at[sv1], add=True)
        kk1 = jnp.minimum(k0 + 3, KT - 1)
        pltpu.sync_copy(gidx.at[c, s, kk1, 0], gv1)
        pltpu.sync_copy(sidx.at[s, kk1, 0], sv1)
        pltpu.async_copy(feat2.at[gv1], rows1, g1)
        return carry

    lax.fori_loop(0, (KT - 1) // 2, pair, 0)
    pltpu.make_async_copy(feat2.at[gv0], rows0, g0).wait()
    pltpu.sync_copy(rows0, acc.at[sv0], add=True)
    pltpu.make_async_copy(feat2.at[gv1], rows1, g1).wait()
    plsc.subcore_barrier()
    pltpu.sync_copy(acc.at[pl.ds(s * RT, RT)],
                    sums.at[c, pl.ds(s * RT, RT)])
    plsc.subcore_barrier()

    pltpu.sync_copy(zrow, acc.at[pl.ds(s * RT, RT)])
    plsc.subcore_barrier()

    def dchunk(k, carry):
        pltpu.sync_copy(sidx.at[s, k, 0], sv0)
        pltpu.sync_copy(ones_v, acc.at[sv0], add=True)
        return carry

    lax.fori_loop(c * 63, 63 + c * 62, dchunk, 0)
    plsc.subcore_barrier()
    pltpu.sync_copy(acc.at[pl.ds(s * RT, RT)],
                    deg.at[c, pl.ds(s * RT, RT)])


_sc_dir = functools.partial(
    pl.kernel,
    out_type=[
        jax.ShapeDtypeStruct((NC, NP, H), jnp.float32),
        jax.ShapeDtypeStruct((NC, NP, H), jnp.float32),
    ],
    mesh=plsc.VectorSubcoreMesh(core_axis_name="c", subcore_axis_name="s"),
    scratch_types=[
        pltpu.VMEM_SHARED((NP, H), jnp.float32),
        pltpu.VMEM((C,), jnp.int32),
        pltpu.VMEM((C,), jnp.int32),
        pltpu.VMEM((C,), jnp.int32),
        pltpu.VMEM((C,), jnp.int32),
        pltpu.VMEM((C, H), jnp.float32),
        pltpu.VMEM((C, H), jnp.float32),
        pltpu.VMEM((C, H), jnp.float32),
        pltpu.SemaphoreType.DMA,
        pltpu.SemaphoreType.DMA,
    ],
)(_sc_dir_body)


R = 2000


def _tc_body(fl, fr, bl, br, di, do_, wa, wb, wc, wd, out):
    sin = 1.0 / jnp.maximum(di[:, 0:1], 1.0)
    so = 1.0 / jnp.maximum(do_[:, 0:1], 1.0)
    acc = jnp.dot(fl[:] * sin, wa[:], preferred_element_type=jnp.float32)
    acc += jnp.dot(fr[:] * sin, wb[:], preferred_element_type=jnp.float32)
    acc += jnp.dot(bl[:] * so, wc[:], preferred_element_type=jnp.float32)
    acc += jnp.dot(br[:] * so, wd[:], preferred_element_type=jnp.float32)
    out[:] = acc


def _tc_mm(fl, fr, bl, br, di, do_, wa, wb, wc, wd):
    sblock = pl.BlockSpec((R, H), lambda i: (i, 0))
    dblock = pl.BlockSpec((R, 16), lambda i: (i, 0))
    wblock = pl.BlockSpec((H, OUT), lambda i: (0, 0))
    return pl.pallas_call(
        _tc_body,
        grid=(N // R,),
        in_specs=[sblock, sblock, sblock, sblock, dblock, dblock,
                  wblock, wblock, wblock, wblock],
        out_specs=pl.BlockSpec((R, OUT), lambda i: (i, 0)),
        out_shape=jax.ShapeDtypeStruct((N, OUT), jnp.float32),
    )(fl, fr, bl, br, di, do_, wa, wb, wc, wd)


def kernel(feat, edge_index, W):
    feat = feat.astype(jnp.float32)
    ei = edge_index.astype(jnp.int32)
    feat2 = jnp.concatenate([feat[:, :H], feat[:, H:]], axis=0)
    offs = (jnp.arange(NC, dtype=jnp.int32) * N)[:, None]
    zrow = jnp.zeros((RT, H), jnp.float32)
    orow = jnp.ones((C, H), jnp.float32)

    outs = []
    for d in range(2):
        gidx = (ei[d][None, :] + offs).reshape(NC, NS, KT, 1, C)
        sidx = ei[1 - d].reshape(NS, KT, 1, C)
        outs.append(_sc_dir(feat2, gidx, sidx, zrow, orow))
    (fsum, fdeg), (bsum, bdeg) = outs

    di = fdeg[0, :N, :16] + fdeg[1, :N, :16]
    do_ = bdeg[0, :N, :16] + bdeg[1, :N, :16]
    Wt = W.astype(jnp.float32).T
    return _tc_mm(fsum[0, :N], fsum[1, :N], bsum[0, :N], bsum[1, :N],
                  di, do_,
                  Wt[0:H], Wt[H:D], Wt[D:D + H], Wt[D + H:2 * D])


# speedup vs baseline: 3.9642x; 1.0806x over previous
import functools

import jax
import jax.numpy as jnp
from jax import lax
from jax.experimental import pallas as pl
from jax.experimental.pallas import tpu as pltpu
from jax.experimental.pallas import tpu_sc as plsc

N = 10000
E = 160000
D = 256
OUT = 256
H = 128
NC = 2
NS = 16
EPT = E // NS
C = 80
KT = EPT // C
NP = 10240
RT = NP // NS
L = 16


KTD = 64


def _sc_dir_body(feat2, gidx, sidx, dsidx, zrow, orow, sums, deg,
                 acc, gv0, sv0, gv1, sv1, rows0, rows1, ones_v, g0, g1):
    c = lax.axis_index("c")
    s = lax.axis_index("s")
    pltpu.sync_copy(orow, ones_v)
    pltpu.sync_copy(zrow, acc.at[pl.ds(s * RT, RT)])
    plsc.subcore_barrier()

    pltpu.sync_copy(gidx.at[c, s, 0, 0], gv0)
    pltpu.sync_copy(sidx.at[s, 0, 0], sv0)
    pltpu.async_copy(feat2.at[gv0], rows0, g0)
    pltpu.sync_copy(gidx.at[c, s, 1, 0], gv1)
    pltpu.sync_copy(sidx.at[s, 1, 0], sv1)
    pltpu.async_copy(feat2.at[gv1], rows1, g1)

    def pair(j, carry):
        k0 = 2 * j
        pltpu.make_async_copy(feat2.at[gv0], rows0, g0).wait()
        pltpu.sync_copy(rows0, acc.at[sv0], add=True)
        kk0 = jnp.minimum(k0 + 2, KT - 1)
        pltpu.sync_copy(gidx.at[c, s, kk0, 0], gv0)
        pltpu.sync_copy(sidx.at[s, kk0, 0], sv0)
        pltpu.async_copy(feat2.at[gv0], rows0, g0)
        pltpu.make_async_copy(feat2.at[gv1], rows1, g1).wait()
        pltpu.sync_copy(rows1, acc.at[sv1], add=True)
        kk1 = jnp.minimum(k0 + 3, KT - 1)
        pltpu.sync_copy(gidx.at[c, s, kk1, 0], gv1)
        pltpu.sync_copy(sidx.at[s, kk1, 0], sv1)
        pltpu.async_copy(feat2.at[gv1], rows1, g1)
        return carry

    lax.fori_loop(0, (KT - 1) // 2, pair, 0)
    pltpu.make_async_copy(feat2.at[gv0], rows0, g0).wait()
    pltpu.sync_copy(rows0, acc.at[sv0], add=True)
    pltpu.make_async_copy(feat2.at[gv1], rows1, g1).wait()
    plsc.subcore_barrier()
    pltpu.sync_copy(acc.at[pl.ds(s * RT, RT)],
                    sums.at[c, pl.ds(s * RT, RT)])
    plsc.subcore_barrier()

    pltpu.sync_copy(zrow, acc.at[pl.ds(s * RT, RT)])
    plsc.subcore_barrier()

    pltpu.async_copy(dsidx.at[c, s, 0, 0], sv0, g0)
    pltpu.async_copy(dsidx.at[c, s, 1, 0], sv1, g1)

    def dpair(j, carry):
        k0 = 2 * j
        pltpu.make_async_copy(dsidx.at[c, s, 0, 0], sv0, g0).wait()
        pltpu.sync_copy(ones_v, acc.at[sv0], add=True)
        kk0 = jnp.minimum(k0 + 2, KTD - 1)
        pltpu.async_copy(dsidx.at[c, s, kk0, 0], sv0, g0)
        pltpu.make_async_copy(dsidx.at[c, s, 0, 0], sv1, g1).wait()
        pltpu.sync_copy(ones_v, acc.at[sv1], add=True)
        kk1 = jnp.minimum(k0 + 3, KTD - 1)
        pltpu.async_copy(dsidx.at[c, s, kk1, 0], sv1, g1)
        return carry

    lax.fori_loop(0, KTD // 2, dpair, 0)
    pltpu.make_async_copy(dsidx.at[c, s, 0, 0], sv0, g0).wait()
    pltpu.make_async_copy(dsidx.at[c, s, 0, 0], sv1, g1).wait()
    plsc.subcore_barrier()
    pltpu.sync_copy(acc.at[pl.ds(s * RT, RT)],
                    deg.at[c, pl.ds(s * RT, RT)])


_sc_dir = functools.partial(
    pl.kernel,
    out_type=[
        jax.ShapeDtypeStruct((NC, NP, H), jnp.float32),
        jax.ShapeDtypeStruct((NC, NP, H), jnp.float32),
    ],
    mesh=plsc.VectorSubcoreMesh(core_axis_name="c", subcore_axis_name="s"),
    scratch_types=[
        pltpu.VMEM_SHARED((NP, H), jnp.float32),
        pltpu.VMEM((C,), jnp.int32),
        pltpu.VMEM((C,), jnp.int32),
        pltpu.VMEM((C,), jnp.int32),
        pltpu.VMEM((C,), jnp.int32),
        pltpu.VMEM((C, H), jnp.float32),
        pltpu.VMEM((C, H), jnp.float32),
        pltpu.VMEM((C, H), jnp.float32),
        pltpu.SemaphoreType.DMA,
        pltpu.SemaphoreType.DMA,
    ],
)(_sc_dir_body)


R = 2000


def _tc_body(fl, fr, bl, br, di, do_, wa, wb, wc, wd, out):
    sin = 1.0 / jnp.maximum(di[:, 0:1], 1.0)
    so = 1.0 / jnp.maximum(do_[:, 0:1], 1.0)
    acc = jnp.dot(fl[:] * sin, wa[:], preferred_element_type=jnp.float32)
    acc += jnp.dot(fr[:] * sin, wb[:], preferred_element_type=jnp.float32)
    acc += jnp.dot(bl[:] * so, wc[:], preferred_element_type=jnp.float32)
    acc += jnp.dot(br[:] * so, wd[:], preferred_element_type=jnp.float32)
    out[:] = acc


def _tc_mm(fl, fr, bl, br, di, do_, wa, wb, wc, wd):
    sblock = pl.BlockSpec((R, H), lambda i: (i, 0))
    dblock = pl.BlockSpec((R, 16), lambda i: (i, 0))
    wblock = pl.BlockSpec((H, OUT), lambda i: (0, 0))
    return pl.pallas_call(
        _tc_body,
        grid=(N // R,),
        in_specs=[sblock, sblock, sblock, sblock, dblock, dblock,
                  wblock, wblock, wblock, wblock],
        out_specs=pl.BlockSpec((R, OUT), lambda i: (i, 0)),
        out_shape=jax.ShapeDtypeStruct((N, OUT), jnp.float32),
    )(fl, fr, bl, br, di, do_, wa, wb, wc, wd)


def kernel(feat, edge_index, W):
    feat = feat.astype(jnp.float32)
    ei = edge_index.astype(jnp.int32)
    feat2 = jnp.concatenate([feat[:, :H], feat[:, H:]], axis=0)
    offs = (jnp.arange(NC, dtype=jnp.int32) * N)[:, None]
    zrow = jnp.zeros((RT, H), jnp.float32)
    orow = jnp.ones((C, H), jnp.float32)

    trash = (N + (jnp.arange(64 * 80 - 5000, dtype=jnp.int32) % (NP - N)))

    outs = []
    for d in range(2):
        gidx = (ei[d][None, :] + offs).reshape(NC, NS, KT, 1, C)
        sidx = ei[1 - d].reshape(NS, KT, 1, C)
        s2 = ei[1 - d].reshape(NS, NC, EPT // NC)
        pad = jnp.broadcast_to(trash[None, None, :], (NS, NC, trash.shape[0]))
        dsidx = jnp.concatenate([s2, pad], axis=2).transpose(1, 0, 2) \
                   .reshape(NC, NS, 64, 1, C)
        outs.append(_sc_dir(feat2, gidx, sidx, dsidx, zrow, orow))
    (fsum, fdeg), (bsum, bdeg) = outs

    di = fdeg[0, :N, :16] + fdeg[1, :N, :16]
    do_ = bdeg[0, :N, :16] + bdeg[1, :N, :16]
    Wt = W.astype(jnp.float32).T
    return _tc_mm(fsum[0, :N], fsum[1, :N], bsum[0, :N], bsum[1, :N],
                  di, do_,
                  Wt[0:H], Wt[H:D], Wt[D:D + H], Wt[D + H:2 * D])
